# Initial kernel scaffold; baseline (speedup 1.0000x reference)
import jax
import jax.numpy as jnp
from jax.experimental import pallas as pl


def kernel(x, edge_index, W1, b1, W2, b2):
    raise NotImplementedError("write your pallas kernel here")


# speedup vs baseline: 19.7786x; 19.7786x over previous
import functools

import jax
import jax.numpy as jnp
from jax import lax
from jax.experimental import pallas as pl
from jax.experimental.pallas import tpu as pltpu
from jax.experimental.pallas import tpu_sc as plsc

N = 10000
DH = 128
E = 320000
NC = 2
NS = 16
NW = NC * NS
EPW = E // NW
K = 80
NCHUNK = EPW // K
NPAD = 10240
DEG_SLICE = NPAD // NS
ROWS_PER_TILE = NPAD // NS
RZ = 16

R = 1000
GRID = N // R

_sc_mesh = plsc.VectorSubcoreMesh(core_axis_name="c", subcore_axis_name="s")


@functools.partial(
    pl.kernel,
    out_type=jax.ShapeDtypeStruct((NC, NPAD), jnp.float32),
    mesh=_sc_mesh,
    scratch_types=[
        pltpu.VMEM((NCHUNK, K), jnp.int32),
        pltpu.VMEM((K,), jnp.float32),
        pltpu.VMEM((DEG_SLICE,), jnp.float32),
        pltpu.VMEM_SHARED((NPAD,), jnp.float32),
    ],
)
def _deg_kernel(dst_hbm, out_hbm, dst_v, ones_v, zbuf, deg_sh):
    c = lax.axis_index("c")
    s = lax.axis_index("s")
    w = c * NS + s
    pltpu.sync_copy(dst_hbm.at[w], dst_v)
    for i in range(K // 16):
        ones_v[pl.ds(i * 16, 16)] = jnp.ones((16,), jnp.float32)
    for i in range(DEG_SLICE // 16):
        zbuf[pl.ds(i * 16, 16)] = jnp.zeros((16,), jnp.float32)
    pltpu.sync_copy(zbuf, deg_sh.at[pl.ds(s * DEG_SLICE, DEG_SLICE)])
    plsc.subcore_barrier()

    def body(j, carry):
        pltpu.sync_copy(ones_v, deg_sh.at[dst_v.at[j]], add=True)
        return carry

    lax.fori_loop(0, NCHUNK, body, 0)
    plsc.subcore_barrier()
    pltpu.sync_copy(deg_sh.at[pl.ds(s * DEG_SLICE, DEG_SLICE)],
                    out_hbm.at[c, pl.ds(s * DEG_SLICE, DEG_SLICE)])


@functools.partial(
    pl.kernel,
    out_type=jax.ShapeDtypeStruct((NC, NPAD, DH), jnp.float32),
    mesh=_sc_mesh,
    scratch_types=[
        pltpu.VMEM((NCHUNK, K), jnp.int32),
        pltpu.VMEM((NCHUNK, K), jnp.int32),
        pltpu.VMEM((K, DH), jnp.float32),
        pltpu.VMEM((RZ, DH), jnp.float32),
        pltpu.VMEM_SHARED((NPAD, DH), jnp.float32),
        pltpu.SemaphoreType.DMA,
    ],
)
def _agg_kernel(g_hbm, src_hbm, dst_hbm, out_hbm,
                src_v, dst_v, rows_v, zbuf, acc_sh, sem):
    c = lax.axis_index("c")
    s = lax.axis_index("s")
    w = c * NS + s
    pltpu.sync_copy(src_hbm.at[w], src_v)
    pltpu.sync_copy(dst_hbm.at[w], dst_v)

    def zrow(i, carry):
        for jj in range(DH // 16):
            zbuf[i, pl.ds(jj * 16, 16)] = jnp.zeros((16,), jnp.float32)
        return carry

    lax.fori_loop(0, RZ, zrow, 0)
    for r in range(ROWS_PER_TILE // RZ):
        pltpu.sync_copy(zbuf, acc_sh.at[pl.ds(s * ROWS_PER_TILE + r * RZ, RZ)])
    plsc.subcore_barrier()

    def body(j, carry):
        pltpu.async_copy(g_hbm.at[src_v.at[j]], rows_v, sem).wait()
        pltpu.sync_copy(rows_v, acc_sh.at[dst_v.at[j]], add=True)
        return carry

    lax.fori_loop(0, NCHUNK, body, 0)
    plsc.subcore_barrier()
    pltpu.sync_copy(acc_sh.at[pl.ds(s * ROWS_PER_TILE, ROWS_PER_TILE)],
                    out_hbm.at[c, pl.ds(s * ROWS_PER_TILE, ROWS_PER_TILE)])


def _dinv(deg_ref):
    return lax.rsqrt(deg_ref[0] + deg_ref[1] + 1.0)


def _tc_pre_body(deg_ref, x_ref, w1_ref, g1_ref):
    g1_ref[...] = _dinv(deg_ref) * jnp.dot(
        x_ref[...], w1_ref[...], preferred_element_type=jnp.float32)


def _tc_mid_body(deg_ref, acc_ref, g1_ref, b1_ref, w2_ref, g2_ref):
    dinv = _dinv(deg_ref)
    a = acc_ref[0] + acc_ref[1] + g1_ref[...]
    h = jnp.maximum(dinv * a + b1_ref[...], 0.0)
    g2_ref[...] = dinv * jnp.dot(h, w2_ref[...],
                                 preferred_element_type=jnp.float32)


def _tc_post_body(deg_ref, acc_ref, g2_ref, b2_ref, out_ref):
    dinv = _dinv(deg_ref)
    a = acc_ref[0] + acc_ref[1] + g2_ref[...]
    out_ref[...] = jnp.maximum(dinv * a + b2_ref[...], 0.0)


_deg_spec = pl.BlockSpec((NC, R, 1), lambda i: (0, i, 0))
_row_spec = pl.BlockSpec((R, DH), lambda i: (i, 0))
_acc_spec = pl.BlockSpec((NC, R, DH), lambda i: (0, i, 0))
_w_spec = pl.BlockSpec((DH, DH), lambda i: (0, 0))
_b_spec = pl.BlockSpec((1, DH), lambda i: (0, 0))

_tc_pre = pl.pallas_call(
    _tc_pre_body,
    grid=(GRID,),
    in_specs=[_deg_spec, _row_spec, _w_spec],
    out_specs=_row_spec,
    out_shape=jax.ShapeDtypeStruct((N, DH), jnp.float32),
)

_tc_mid = pl.pallas_call(
    _tc_mid_body,
    grid=(GRID,),
    in_specs=[_deg_spec, _acc_spec, _row_spec, _b_spec, _w_spec],
    out_specs=_row_spec,
    out_shape=jax.ShapeDtypeStruct((N, DH), jnp.float32),
)

_tc_post = pl.pallas_call(
    _tc_post_body,
    grid=(GRID,),
    in_specs=[_deg_spec, _acc_spec, _row_spec, _b_spec],
    out_specs=_row_spec,
    out_shape=jax.ShapeDtypeStruct((N, DH), jnp.float32),
)


def kernel(x, edge_index, W1, b1, W2, b2):
    src = edge_index[0].reshape(NW, NCHUNK, K)
    dst = edge_index[1].reshape(NW, NCHUNK, K)
    b1r = b1.reshape(1, DH)
    b2r = b2.reshape(1, DH)

    degp = _deg_kernel(dst).reshape(NC, NPAD, 1)
    g1 = _tc_pre(degp, x, W1)
    acc1 = _agg_kernel(g1, src, dst)
    g2 = _tc_mid(degp, acc1, g1, b1r, W2)
    acc2 = _agg_kernel(g2, src, dst)
    return _tc_post(degp, acc2, g2, b2r)


# speedup vs baseline: 28.8947x; 1.4609x over previous
import functools

import jax
import jax.numpy as jnp
from jax import lax
from jax.experimental import pallas as pl
from jax.experimental.pallas import tpu as pltpu
from jax.experimental.pallas import tpu_sc as plsc

N = 10000
DH = 128
E = 320000
NC = 2
NS = 16
NW = NC * NS
EPW = E // NW
K = 80
NCHUNK = EPW // K
SSTG = 25
NSTG = NCHUNK // SSTG
NPAIR = (SSTG - 1) // 2
NPAD = 10240
DEG_SLICE = NPAD // NS
ROWS_PER_TILE = NPAD // NS
RZ = 16

R = 1000
GRID = N // R

_sc_mesh = plsc.VectorSubcoreMesh(core_axis_name="c", subcore_axis_name="s")


@functools.partial(
    pl.kernel,
    out_type=jax.ShapeDtypeStruct((NC, NPAD), jnp.float32),
    mesh=_sc_mesh,
    scratch_types=[
        pltpu.VMEM((NSTG, SSTG, K), jnp.int32),
        pltpu.VMEM((K,), jnp.float32),
        pltpu.VMEM((DEG_SLICE,), jnp.float32),
        pltpu.VMEM_SHARED((NPAD,), jnp.float32),
    ],
)
def _deg_kernel(dst_hbm, out_hbm, dst_v, ones_v, zbuf, deg_sh):
    c = lax.axis_index("c")
    s = lax.axis_index("s")
    w = c * NS + s
    pltpu.sync_copy(dst_hbm.at[w], dst_v)
    for i in range(K // 16):
        ones_v[pl.ds(i * 16, 16)] = jnp.ones((16,), jnp.float32)
    for i in range(DEG_SLICE // 16):
        zbuf[pl.ds(i * 16, 16)] = jnp.zeros((16,), jnp.float32)
    pltpu.sync_copy(zbuf, deg_sh.at[pl.ds(s * DEG_SLICE, DEG_SLICE)])
    plsc.subcore_barrier()

    def body(j, carry):
        pltpu.sync_copy(ones_v, deg_sh.at[dst_v.at[j // SSTG, j % SSTG]], add=True)
        return carry

    lax.fori_loop(0, NCHUNK, body, 0)
    plsc.subcore_barrier()
    pltpu.sync_copy(deg_sh.at[pl.ds(s * DEG_SLICE, DEG_SLICE)],
                    out_hbm.at[c, pl.ds(s * DEG_SLICE, DEG_SLICE)])


@functools.partial(
    pl.kernel,
    out_type=jax.ShapeDtypeStruct((NC, NPAD, DH), jnp.float32),
    mesh=_sc_mesh,
    scratch_types=[
        pltpu.VMEM((SSTG, K), jnp.int32),
        pltpu.VMEM((SSTG, K), jnp.int32),
        pltpu.VMEM((K, DH), jnp.float32),
        pltpu.VMEM((K, DH), jnp.float32),
        pltpu.VMEM((RZ, DH), jnp.float32),
        pltpu.VMEM_SHARED((NPAD, DH), jnp.float32),
        pltpu.SemaphoreType.DMA,
        pltpu.SemaphoreType.DMA,
    ],
)
def _agg_kernel(g_hbm, src_hbm, dst_hbm, out_hbm,
                src_v, dst_v, rows0, rows1, zbuf, acc_sh, sem0, sem1):
    c = lax.axis_index("c")
    s = lax.axis_index("s")
    w = c * NS + s

    def zrow(i, carry):
        for jj in range(DH // 16):
            zbuf[i, pl.ds(jj * 16, 16)] = jnp.zeros((16,), jnp.float32)
        return carry

    lax.fori_loop(0, RZ, zrow, 0)
    for r in range(ROWS_PER_TILE // RZ):
        pltpu.sync_copy(zbuf, acc_sh.at[pl.ds(s * ROWS_PER_TILE + r * RZ, RZ)])
    plsc.subcore_barrier()

    def start(j, buf, sem):
        pltpu.async_copy(g_hbm.at[src_v.at[j]], buf, sem)

    def wait(buf, sem):
        pltpu.make_async_copy(g_hbm.at[pl.ds(0, K)], buf, sem).wait()

    def scat(j, buf):
        pltpu.sync_copy(buf, acc_sh.at[dst_v.at[j]], add=True)

    def stage(st, carry):
        pltpu.sync_copy(src_hbm.at[w, st], src_v)
        pltpu.sync_copy(dst_hbm.at[w, st], dst_v)
        start(0, rows0, sem0)

        def pair(i, carry2):
            j = 2 * i
            start(j + 1, rows1, sem1)
            wait(rows0, sem0)
            scat(j, rows0)
            start(j + 2, rows0, sem0)
            wait(rows1, sem1)
            scat(j + 1, rows1)
            return carry2

        lax.fori_loop(0, NPAIR, pair, 0)
        wait(rows0, sem0)
        scat(SSTG - 1, rows0)
        return carry

    lax.fori_loop(0, NSTG, stage, 0)
    plsc.subcore_barrier()
    pltpu.sync_copy(acc_sh.at[pl.ds(s * ROWS_PER_TILE, ROWS_PER_TILE)],
                    out_hbm.at[c, pl.ds(s * ROWS_PER_TILE, ROWS_PER_TILE)])


def _dinv(deg_ref):
    return lax.rsqrt(deg_ref[0] + deg_ref[1] + 1.0)


def _tc_pre_body(deg_ref, x_ref, w1_ref, g1_ref):
    g1_ref[...] = _dinv(deg_ref) * jnp.dot(
        x_ref[...], w1_ref[...], preferred_element_type=jnp.float32)


def _tc_mid_body(deg_ref, acc_ref, g1_ref, b1_ref, w2_ref, g2_ref):
    dinv = _dinv(deg_ref)
    a = acc_ref[0] + acc_ref[1] + g1_ref[...]
    h = jnp.maximum(dinv * a + b1_ref[...], 0.0)
    g2_ref[...] = dinv * jnp.dot(h, w2_ref[...],
                                 preferred_element_type=jnp.float32)


def _tc_post_body(deg_ref, acc_ref, g2_ref, b2_ref, out_ref):
    dinv = _dinv(deg_ref)
    a = acc_ref[0] + acc_ref[1] + g2_ref[...]
    out_ref[...] = jnp.maximum(dinv * a + b2_ref[...], 0.0)


_deg_spec = pl.BlockSpec((NC, R, 1), lambda i: (0, i, 0))
_row_spec = pl.BlockSpec((R, DH), lambda i: (i, 0))
_acc_spec = pl.BlockSpec((NC, R, DH), lambda i: (0, i, 0))
_w_spec = pl.BlockSpec((DH, DH), lambda i: (0, 0))
_b_spec = pl.BlockSpec((1, DH), lambda i: (0, 0))

_tc_pre = pl.pallas_call(
    _tc_pre_body,
    grid=(GRID,),
    in_specs=[_deg_spec, _row_spec, _w_spec],
    out_specs=_row_spec,
    out_shape=jax.ShapeDtypeStruct((N, DH), jnp.float32),
)

_tc_mid = pl.pallas_call(
    _tc_mid_body,
    grid=(GRID,),
    in_specs=[_deg_spec, _acc_spec, _row_spec, _b_spec, _w_spec],
    out_specs=_row_spec,
    out_shape=jax.ShapeDtypeStruct((N, DH), jnp.float32),
)

_tc_post = pl.pallas_call(
    _tc_post_body,
    grid=(GRID,),
    in_specs=[_deg_spec, _acc_spec, _row_spec, _b_spec],
    out_specs=_row_spec,
    out_shape=jax.ShapeDtypeStruct((N, DH), jnp.float32),
)


def kernel(x, edge_index, W1, b1, W2, b2):
    src = edge_index[0].reshape(NW, NSTG, SSTG, K)
    dst = edge_index[1].reshape(NW, NSTG, SSTG, K)
    b1r = b1.reshape(1, DH)
    b2r = b2.reshape(1, DH)

    degp = _deg_kernel(dst).reshape(NC, NPAD, 1)
    g1 = _tc_pre(degp, x, W1)
    acc1 = _agg_kernel(g1, src, dst)
    g2 = _tc_mid(degp, acc1, g1, b1r, W2)
    acc2 = _agg_kernel(g2, src, dst)
    return _tc_post(degp, acc2, g2, b2r)
